# baseline (speedup 1.0000x reference)
import jax
import jax.numpy as jnp
from jax import lax
from jax.experimental import pallas as pl
from jax.experimental.pallas import tpu as pltpu
from jax.experimental.pallas import tpu_sc as plsc

_ROWS = 16
_D = 2048
_L = 16
_NCHUNK = _D // _L


def _pooler_body(hs_hbm, lens_hbm, out_hbm, lens_v, row_v):
    c = lax.axis_index("c")
    s = lax.axis_index("s")
    wid = s * 2 + c

    @pl.when(wid < _ROWS)
    def _():
        pltpu.sync_copy(lens_hbm, lens_v)
        lens = lens_v[...]
        cum = jnp.int32(0)
        my_cum = jnp.int32(0)
        for i in range(_ROWS):
            cum = cum + lens[i]
            my_cum = jnp.where(wid == i, cum, my_cum)
        my_idx = my_cum - 1
        my_idx = jnp.where(my_idx < 0, my_idx + hs_hbm.shape[0], my_idx)

        pltpu.sync_copy(hs_hbm.at[my_idx], row_v)

        acc = jnp.zeros((_L,), jnp.float32)
        for k in range(_NCHUNK):
            v = row_v[pl.ds(k * _L, _L)]
            acc = acc + v * v
        ss = jnp.float32(0.0)
        for i in range(_L):
            ss = ss + acc[i]

        x = jnp.maximum(ss, jnp.float32(1e-24))
        bits = lax.bitcast_convert_type(x, jnp.int32)
        y = lax.bitcast_convert_type(
            jnp.int32(0x5F3759DF) - lax.shift_right_arithmetic(bits, 1),
            jnp.float32,
        )
        half_x = jnp.float32(0.5) * x
        for _ in range(3):
            y = y * (jnp.float32(1.5) - half_x * y * y)
        scale = jnp.broadcast_to(y, (_L,))

        for k in range(_NCHUNK):
            sl = pl.ds(k * _L, _L)
            row_v[sl] = row_v[sl] * scale

        pltpu.sync_copy(row_v, out_hbm.at[wid])


def kernel(hidden_states, extend_seq_lens):
    mesh = plsc.VectorSubcoreMesh(core_axis_name="c", subcore_axis_name="s")
    return pl.kernel(
        _pooler_body,
        out_type=jax.ShapeDtypeStruct((_ROWS, _D), jnp.float32),
        mesh=mesh,
        scratch_types=[
            pltpu.VMEM((_L,), jnp.int32),
            pltpu.VMEM((_D,), jnp.float32),
        ],
    )(hidden_states, extend_seq_lens)


# speedup vs baseline: 1.0049x; 1.0049x over previous
import jax
import jax.numpy as jnp
from jax import lax
from jax.experimental import pallas as pl
from jax.experimental.pallas import tpu as pltpu
from jax.experimental.pallas import tpu_sc as plsc

_ROWS = 16
_D = 2048
_L = 16
_NCHUNK = _D // _L


def _pooler_body(hs_hbm, lens_hbm, out_hbm, lens_v, row_v):
    c = lax.axis_index("c")
    s = lax.axis_index("s")
    wid = s * 2 + c

    @pl.when(wid < _ROWS)
    def _():
        pltpu.sync_copy(lens_hbm, lens_v)
        lens = lens_v[...]
        cum = jnp.int32(0)
        my_cum = jnp.int32(0)
        for i in range(_ROWS):
            cum = cum + lens[i]
            my_cum = jnp.where(wid == i, cum, my_cum)
        my_idx = my_cum - 1
        my_idx = jnp.where(my_idx < 0, my_idx + hs_hbm.shape[0], my_idx)

        pltpu.sync_copy(hs_hbm.at[my_idx], row_v)

        def ss_body(k, acc):
            v = row_v[pl.ds(k * _L, _L)]
            return acc + v * v

        acc = lax.fori_loop(0, _NCHUNK, ss_body, jnp.zeros((_L,), jnp.float32))
        ss = jnp.float32(0.0)
        for i in range(_L):
            ss = ss + acc[i]

        x = jnp.maximum(ss, jnp.float32(1e-24))
        bits = lax.bitcast_convert_type(x, jnp.int32)
        y = lax.bitcast_convert_type(
            jnp.int32(0x5F3759DF) - lax.shift_right_arithmetic(bits, 1),
            jnp.float32,
        )
        half_x = jnp.float32(0.5) * x
        for _ in range(3):
            y = y * (jnp.float32(1.5) - half_x * y * y)
        scale = jnp.broadcast_to(y, (_L,))

        def scale_body(k, _):
            sl = pl.ds(k * _L, _L)
            row_v[sl] = row_v[sl] * scale
            return 0

        lax.fori_loop(0, _NCHUNK, scale_body, 0)

        pltpu.sync_copy(row_v, out_hbm.at[wid])


def kernel(hidden_states, extend_seq_lens):
    mesh = plsc.VectorSubcoreMesh(core_axis_name="c", subcore_axis_name="s")
    return pl.kernel(
        _pooler_body,
        out_type=jax.ShapeDtypeStruct((_ROWS, _D), jnp.float32),
        mesh=mesh,
        scratch_types=[
            pltpu.VMEM((_L,), jnp.int32),
            pltpu.VMEM((_D,), jnp.float32),
        ],
    )(hidden_states, extend_seq_lens)


# speedup vs baseline: 1.0740x; 1.0688x over previous
import jax
import jax.numpy as jnp
from jax import lax
from jax.experimental import pallas as pl
from jax.experimental.pallas import tpu as pltpu
from jax.experimental.pallas import tpu_sc as plsc

_ROWS = 16
_D = 2048
_L = 16
_NCHUNK = _D // _L


def _pooler_body(hs_hbm, lens_hbm, out_hbm, lens_v, row_v):
    c = lax.axis_index("c")
    s = lax.axis_index("s")
    wid = s + c * 0

    @pl.when(wid < _ROWS)
    def _():
        pltpu.sync_copy(lens_hbm, lens_v)
        lens = lens_v[...]
        cum = jnp.int32(0)
        my_cum = jnp.int32(0)
        for i in range(_ROWS):
            cum = cum + lens[i]
            my_cum = jnp.where(wid == i, cum, my_cum)
        my_idx = my_cum - 1
        my_idx = jnp.where(my_idx < 0, my_idx + hs_hbm.shape[0], my_idx)

        pltpu.sync_copy(hs_hbm.at[my_idx], row_v)

        def ss_body(k, acc):
            v = row_v[pl.ds(k * _L, _L)]
            return acc + v * v

        acc = lax.fori_loop(0, _NCHUNK, ss_body, jnp.zeros((_L,), jnp.float32))
        ss = jnp.float32(0.0)
        for i in range(_L):
            ss = ss + acc[i]

        x = jnp.maximum(ss, jnp.float32(1e-24))
        bits = lax.bitcast_convert_type(x, jnp.int32)
        y = lax.bitcast_convert_type(
            jnp.int32(0x5F3759DF) - lax.shift_right_arithmetic(bits, 1),
            jnp.float32,
        )
        half_x = jnp.float32(0.5) * x
        for _ in range(3):
            y = y * (jnp.float32(1.5) - half_x * y * y)
        scale = jnp.broadcast_to(y, (_L,))

        def scale_body(k, _):
            sl = pl.ds(k * _L, _L)
            row_v[sl] = row_v[sl] * scale
            return 0

        lax.fori_loop(0, _NCHUNK, scale_body, 0)

        pltpu.sync_copy(row_v, out_hbm.at[wid])


def kernel(hidden_states, extend_seq_lens):
    mesh = plsc.VectorSubcoreMesh(core_axis_name="c", subcore_axis_name="s", num_cores=1)
    return pl.kernel(
        _pooler_body,
        out_type=jax.ShapeDtypeStruct((_ROWS, _D), jnp.float32),
        mesh=mesh,
        scratch_types=[
            pltpu.VMEM((_L,), jnp.int32),
            pltpu.VMEM((_D,), jnp.float32),
        ],
    )(hidden_states, extend_seq_lens)
